# baseline (speedup 1.0000x reference)
import jax
import jax.numpy as jnp
from jax.experimental import pallas as pl
from jax.experimental.pallas import tpu as pltpu


def _body(start_ref, kc_ref, vc_ref, key_ref, val_ref, ko_ref, vo_ref):
    T = kc_ref.shape[1]
    NUM = key_ref.shape[1]
    start = start_ref[0]

    row = jax.lax.broadcasted_iota(jnp.int32, (T, NUM), 0)
    col = jax.lax.broadcasted_iota(jnp.int32, (T, NUM), 1)
    j = row - start
    j = jnp.where(j < 0, j + T, j)
    onehot = (j == col).astype(jnp.float32)
    mask = (j < NUM)[:, :1]

    sub_k = jnp.dot(onehot, key_ref[0], preferred_element_type=jnp.float32)
    ko_ref[0] = jnp.where(mask, sub_k, kc_ref[0])
    sub_v = jnp.dot(onehot, val_ref[0], preferred_element_type=jnp.float32)
    vo_ref[0] = jnp.where(mask, sub_v, vc_ref[0])


def kernel(key, value, k_cache, v_cache, input_pos):
    B, G, NUM, H = key.shape
    T = k_cache.shape[2]
    BG = B * G

    key_r = key.reshape(BG, NUM, H)
    val_r = value.reshape(BG, NUM, H)
    kc_r = k_cache.reshape(BG, T, H)
    vc_r = v_cache.reshape(BG, T, H)
    start = (jnp.asarray(input_pos, jnp.int32) % T).reshape(1)

    grid_spec = pltpu.PrefetchScalarGridSpec(
        num_scalar_prefetch=1,
        grid=(BG,),
        in_specs=[
            pl.BlockSpec((1, T, H), lambda i, s: (i, 0, 0)),
            pl.BlockSpec((1, T, H), lambda i, s: (i, 0, 0)),
            pl.BlockSpec((1, NUM, H), lambda i, s: (i, 0, 0)),
            pl.BlockSpec((1, NUM, H), lambda i, s: (i, 0, 0)),
        ],
        out_specs=[
            pl.BlockSpec((1, T, H), lambda i, s: (i, 0, 0)),
            pl.BlockSpec((1, T, H), lambda i, s: (i, 0, 0)),
        ],
    )
    ko, vo = pl.pallas_call(
        _body,
        grid_spec=grid_spec,
        out_shape=[jax.ShapeDtypeStruct((BG, T, H), jnp.float32)] * 2,
    )(start, kc_r, vc_r, key_r, val_r)
    return ko.reshape(B, G, T, H), vo.reshape(B, G, T, H)
